# baseline (speedup 1.0000x reference)
import functools

import jax
import jax.numpy as jnp
from jax import lax
from jax.experimental import pallas as pl
from jax.experimental.pallas import tpu as pltpu
from jax.experimental.pallas import tpu_sc as plsc

EPS = 1e-12

_NUM_CORES = 2
_NUM_SUBCORES = 16
_NUM_WORKERS = _NUM_CORES * _NUM_SUBCORES

_L = 16
_CHUNK = 16
_RSUB = 8
_BLOCK_ROWS = 1024
_N_SC = 7168


def _sc_gather(table, idx_flat):
    n, = idx_flat.shape
    h = table.shape[1]
    rows_per_worker = n // _NUM_WORKERS
    chunk = 32
    n_chunks = rows_per_worker // chunk
    mesh = plsc.VectorSubcoreMesh(
        core_axis_name="c", subcore_axis_name="s",
        num_cores=_NUM_CORES, num_subcores=_NUM_SUBCORES)

    @functools.partial(
        pl.kernel,
        mesh=mesh,
        out_type=jax.ShapeDtypeStruct((n, h), table.dtype),
        scratch_types=[
            pltpu.VMEM((rows_per_worker,), jnp.int32),
            pltpu.VMEM((chunk, h), table.dtype),
            pltpu.VMEM((chunk, h), table.dtype),
            pltpu.SemaphoreType.DMA,
            pltpu.SemaphoreType.DMA,
        ],
    )
    def gather_kernel(table_hbm, idx_hbm, out_hbm, idx_v, rows_a, rows_b,
                      sem_a, sem_b):
        wid = lax.axis_index("s") * _NUM_CORES + lax.axis_index("c")
        base = wid * rows_per_worker
        pltpu.sync_copy(idx_hbm.at[pl.ds(base, rows_per_worker)], idx_v)
        pltpu.async_copy(table_hbm.at[idx_v.at[pl.ds(0, chunk)]], rows_a,
                         sem_a)

        def body(i, carry):
            even = lax.rem(i, 2) == 0
            more = i + 1 < n_chunks

            @pl.when(jnp.logical_and(even, more))
            def _():
                pltpu.async_copy(
                    table_hbm.at[idx_v.at[pl.ds((i + 1) * chunk, chunk)]],
                    rows_b, sem_b)

            @pl.when(jnp.logical_and(jnp.logical_not(even), more))
            def _():
                pltpu.async_copy(
                    table_hbm.at[idx_v.at[pl.ds((i + 1) * chunk, chunk)]],
                    rows_a, sem_a)

            @pl.when(even)
            def _():
                pltpu.make_async_copy(table_hbm.at[pl.ds(0, chunk)], rows_a,
                                      sem_a).wait()
                pltpu.sync_copy(rows_a, out_hbm.at[pl.ds(base + i * chunk,
                                                         chunk)])

            @pl.when(jnp.logical_not(even))
            def _():
                pltpu.make_async_copy(table_hbm.at[pl.ds(0, chunk)], rows_b,
                                      sem_b).wait()
                pltpu.sync_copy(rows_b, out_hbm.at[pl.ds(base + i * chunk,
                                                         chunk)])

            return carry

        lax.fori_loop(0, n_chunks, body, 0)

    return gather_kernel(table, idx_flat)


def _rsqrt_vec(v):
    bits = lax.bitcast_convert_type(v, jnp.int32)
    y = lax.bitcast_convert_type(
        jnp.int32(0x5F3759DF) - lax.shift_right_logical(bits, 1),
        jnp.float32)
    for _ in range(3):
        y = y * (1.5 - 0.5 * v * y * y)
    return y


def _lane_take(v, idx):
    dn = lax.GatherDimensionNumbers(offset_dims=(), collapsed_slice_dims=(0,),
                                    start_index_map=(0,))
    return lax.gather(v, idx[:, None], dn, slice_sizes=(1,),
                      mode=lax.GatherScatterMode.PROMISE_IN_BOUNDS)


def _allreduce_sum(v):
    ix = lax.iota(jnp.int32, _L)
    for step in (1, 2, 4, 8):
        v = v + _lane_take(v, jnp.bitwise_xor(ix, step))
    return v


def _fused_sc(embeds2, idx_flat, tid_flat, pos_table, type_table, w1, b1,
              n_rows):
    n, h = embeds2.shape
    rows_per_worker = n_rows // _NUM_WORKERS
    n_chunks = rows_per_worker // _CHUNK
    nvec = h // _L
    mesh = plsc.VectorSubcoreMesh(
        core_axis_name="c", subcore_axis_name="s",
        num_cores=_NUM_CORES, num_subcores=_NUM_SUBCORES)

    @functools.partial(
        pl.kernel,
        mesh=mesh,
        out_type=jax.ShapeDtypeStruct((n_rows, h), jnp.float32),
        scratch_types=[
            pltpu.VMEM((rows_per_worker,), jnp.int32),
            pltpu.VMEM((rows_per_worker,), jnp.int32),
            pltpu.VMEM((h,), jnp.float32),
            pltpu.VMEM((h,), jnp.float32),
            pltpu.VMEM((h,), jnp.float32),
            pltpu.VMEM((h,), jnp.float32),
            pltpu.VMEM((_CHUNK, h), jnp.float32),
            pltpu.VMEM((_CHUNK, h), jnp.float32),
            pltpu.VMEM((_CHUNK, h), jnp.float32),
            pltpu.VMEM((_CHUNK, h), jnp.float32),
            pltpu.VMEM((_CHUNK, h), jnp.float32),
            pltpu.VMEM((_CHUNK, h), jnp.float32),
            pltpu.SemaphoreType.DMA,
            pltpu.SemaphoreType.DMA,
            pltpu.SemaphoreType.DMA,
            pltpu.SemaphoreType.DMA,
            pltpu.SemaphoreType.DMA,
            pltpu.SemaphoreType.DMA,
            pltpu.SemaphoreType.DMA,
            pltpu.SemaphoreType.DMA,
            pltpu.SemaphoreType.DMA,
        ],
    )
    def fused_kernel(emb_hbm, idx_hbm, tid_hbm, table_hbm, type_hbm,
                     w_hbm, b_hbm, out_hbm,
                     idx_v, tid_v, w_v, b_v, t0_v, d_v,
                     emb0, emb1, emb2, pos0, pos1, pos2,
                     se0, se1, se2, sp0, sp1, sp2, so0, so1, so2):
        embs = (emb0, emb1, emb2)
        poss = (pos0, pos1, pos2)
        sems_e = (se0, se1, se2)
        sems_p = (sp0, sp1, sp2)
        sems_o = (so0, so1, so2)

        wid = lax.axis_index("s") * _NUM_CORES + lax.axis_index("c")
        base = wid * rows_per_worker

        pltpu.sync_copy(idx_hbm.at[pl.ds(base, rows_per_worker)], idx_v)
        pltpu.sync_copy(tid_hbm.at[pl.ds(base, rows_per_worker)], tid_v)
        pltpu.sync_copy(w_hbm, w_v)
        pltpu.sync_copy(b_hbm, b_v)
        pltpu.sync_copy(type_hbm.at[0], t0_v)
        pltpu.sync_copy(type_hbm.at[1], d_v)

        def mkdiff(c, carry):
            sl = pl.ds(c * _L, _L)
            d_v[sl] = d_v[sl] - t0_v[sl]
            return carry
        lax.fori_loop(0, nvec, mkdiff, 0)

        def issue_in(k, ring):
            pltpu.async_copy(emb_hbm.at[pl.ds(base + k * _CHUNK, _CHUNK)],
                             embs[ring], sems_e[ring])
            pltpu.async_copy(
                table_hbm.at[idx_v.at[pl.ds(k * _CHUNK, _CHUNK)]],
                poss[ring], sems_p[ring])

        def wait_in(ring):
            pltpu.make_async_copy(emb_hbm.at[pl.ds(0, _CHUNK)], embs[ring],
                                  sems_e[ring]).wait()
            pltpu.make_async_copy(emb_hbm.at[pl.ds(0, _CHUNK)], poss[ring],
                                  sems_p[ring]).wait()

        def wait_out(ring):
            pltpu.make_async_copy(poss[ring], out_hbm.at[pl.ds(base, _CHUNK)],
                                  sems_o[ring]).wait()

        def compute_sub(k, emb_b, pos_b, sub):
            r0 = sub * _RSUB
            rowbase = k * _CHUNK + r0
            tids16 = tid_v[pl.ds(k * _CHUNK, _L)].astype(jnp.float32)
            tidf = []
            for r in range(_RSUB):
                tidf.append(_lane_take(
                    tids16, jnp.full((_L,), r0 + r, jnp.int32)))

            zero = jnp.zeros((_L,), jnp.float32)

            def p1(c, carry):
                accs = list(carry)
                sl = pl.ds(c * _L, _L)
                t0c = t0_v[sl]
                dc = d_v[sl]
                for r in range(_RSUB):
                    e = emb_b[r0 + r, sl] + pos_b[r0 + r, sl]
                    e = e + t0c + tidf[r] * dc
                    accs[r] = accs[r] + e
                    accs[_RSUB + r] = accs[_RSUB + r] + e * e
                    emb_b[r0 + r, sl] = e
                return tuple(accs)

            carry = tuple([zero] * (2 * _RSUB))
            carry = lax.fori_loop(0, nvec, p1, carry)

            inv_h = jnp.float32(1.0 / h)
            k1s, k2s = [], []
            for r in range(_RSUB):
                u = _allreduce_sum(carry[r]) * inv_h
                sq = _allreduce_sum(carry[_RSUB + r]) * inv_h
                var = sq - u * u
                k1 = _rsqrt_vec(var + jnp.float32(EPS))
                k2 = u * k1
                k1s.append(k1)
                k2s.append(k2)

            def p2(c, carry2):
                sl = pl.ds(c * _L, _L)
                wc = w_v[sl]
                bc = b_v[sl]
                for r in range(_RSUB):
                    e = emb_b[r0 + r, sl]
                    x = e * carry2[r] - carry2[_RSUB + r]
                    pos_b[r0 + r, sl] = x * wc + bc
                return carry2

            lax.fori_loop(0, nvec, p2, tuple(k1s) + tuple(k2s))

        issue_in(0, 0)

        def body(k, carry):
            for j in range(3):
                @pl.when(lax.rem(k, 3) == j)
                def _(j=j):
                    nxt = (j + 1) % 3

                    @pl.when(k < n_chunks - 1)
                    def _():
                        @pl.when(k >= 2)
                        def _():
                            wait_out(nxt)
                        issue_in(k + 1, nxt)

                    wait_in(j)
                    compute_sub(k, embs[j], poss[j], 0)
                    compute_sub(k, embs[j], poss[j], 1)
                    pltpu.async_copy(
                        poss[j],
                        out_hbm.at[pl.ds(base + k * _CHUNK, _CHUNK)],
                        sems_o[j])
            return carry

        lax.fori_loop(0, n_chunks, body, 0)
        for j in range(3):
            wait_out(j)

    return fused_kernel(embeds2, idx_flat, tid_flat, pos_table, type_table,
                        w1, b1)


def _tc_fused_ln(embeds2, pos_j, tids3, type_table, w2, b2,
                 off_blocks, grid_blocks):
    n, h = embeds2.shape
    br = _BLOCK_ROWS

    def body(emb_ref, pos_ref, tid_ref, tt_ref, w_ref, b_ref, out_ref):
        e = emb_ref[...] + pos_ref[...]
        t = tid_ref[0, 0, :].reshape(br, 1)
        te = jnp.where(t == 1, tt_ref[1, :][None, :], tt_ref[0, :][None, :])
        e = e + te
        u = jnp.mean(e, axis=-1, keepdims=True)
        d = e - u
        s = jnp.mean(d * d, axis=-1, keepdims=True)
        x = d * lax.rsqrt(s + EPS)
        out_ref[...] = w_ref[...] * x + b_ref[...]

    return pl.pallas_call(
        body,
        grid=(grid_blocks,),
        in_specs=[
            pl.BlockSpec((br, h), lambda i: (off_blocks + i, 0)),
            pl.BlockSpec((br, h), lambda i: (i, 0)),
            pl.BlockSpec((1, 1, br), lambda i: (off_blocks + i, 0, 0)),
            pl.BlockSpec((2, h), lambda i: (0, 0)),
            pl.BlockSpec((1, h), lambda i: (0, 0)),
            pl.BlockSpec((1, h), lambda i: (0, 0)),
        ],
        out_specs=pl.BlockSpec((br, h), lambda i: (off_blocks + i, 0)),
        out_shape=jax.ShapeDtypeStruct((n, h), jnp.float32),
    )(embeds2, pos_j, tids3, type_table, w2, b2)


def kernel(inputs_embeds, token_type_ids, position_ids, pos_table,
           type_table, ln_weight, ln_bias):
    b, s, h = inputs_embeds.shape
    n = b * s
    embeds2 = inputs_embeds.reshape(n, h)
    pos_flat = position_ids.reshape(n).astype(jnp.int32)
    tid_flat = token_type_ids.reshape(n).astype(jnp.int32)
    tids3 = token_type_ids.reshape(n // _BLOCK_ROWS, 1, _BLOCK_ROWS).astype(jnp.int32)
    w2 = ln_weight.reshape(1, h)
    b2 = ln_bias.reshape(1, h)

    n_tc = n - _N_SC
    off_blocks = _N_SC // _BLOCK_ROWS

    pos_tc = _sc_gather(pos_table, pos_flat[_N_SC:])
    out_sc = _fused_sc(embeds2, pos_flat, tid_flat, pos_table, type_table,
                       ln_weight, ln_bias, _N_SC)
    tc_full = _tc_fused_ln(embeds2, pos_tc, tids3, type_table, w2, b2,
                           off_blocks, n_tc // _BLOCK_ROWS)
    out2 = lax.dynamic_update_slice(tc_full, out_sc, (0, 0))
    return out2.reshape(b, s, h)


# speedup vs baseline: 1.0338x; 1.0338x over previous
import functools

import jax
import jax.numpy as jnp
from jax import lax
from jax.experimental import pallas as pl
from jax.experimental.pallas import tpu as pltpu
from jax.experimental.pallas import tpu_sc as plsc

EPS = 1e-12

_NUM_CORES = 2
_NUM_SUBCORES = 16
_NUM_WORKERS = _NUM_CORES * _NUM_SUBCORES


def _sc_gather(table, idx_flat):
    n, = idx_flat.shape
    h = table.shape[1]
    rows_per_worker = n // _NUM_WORKERS
    chunk = 32
    n_chunks = rows_per_worker // chunk
    mesh = plsc.VectorSubcoreMesh(
        core_axis_name="c", subcore_axis_name="s",
        num_cores=_NUM_CORES, num_subcores=_NUM_SUBCORES)

    @functools.partial(
        pl.kernel,
        mesh=mesh,
        out_type=jax.ShapeDtypeStruct((n, h), table.dtype),
        scratch_types=[
            pltpu.VMEM((rows_per_worker,), jnp.int32),
            pltpu.VMEM((chunk, h), table.dtype),
            pltpu.VMEM((chunk, h), table.dtype),
            pltpu.SemaphoreType.DMA,
            pltpu.SemaphoreType.DMA,
        ],
    )
    def gather_kernel(table_hbm, idx_hbm, out_hbm, idx_v, rows_a, rows_b,
                      sem_a, sem_b):
        wid = lax.axis_index("s") * _NUM_CORES + lax.axis_index("c")
        base = wid * rows_per_worker
        pltpu.sync_copy(idx_hbm.at[pl.ds(base, rows_per_worker)], idx_v)
        pltpu.async_copy(table_hbm.at[idx_v.at[pl.ds(0, chunk)]], rows_a,
                         sem_a)

        def body(i, carry):
            even = lax.rem(i, 2) == 0
            more = i + 1 < n_chunks

            @pl.when(jnp.logical_and(even, more))
            def _():
                pltpu.async_copy(
                    table_hbm.at[idx_v.at[pl.ds((i + 1) * chunk, chunk)]],
                    rows_b, sem_b)

            @pl.when(jnp.logical_and(jnp.logical_not(even), more))
            def _():
                pltpu.async_copy(
                    table_hbm.at[idx_v.at[pl.ds((i + 1) * chunk, chunk)]],
                    rows_a, sem_a)

            @pl.when(even)
            def _():
                pltpu.make_async_copy(table_hbm.at[pl.ds(0, chunk)], rows_a,
                                      sem_a).wait()
                pltpu.sync_copy(rows_a, out_hbm.at[pl.ds(base + i * chunk,
                                                         chunk)])

            @pl.when(jnp.logical_not(even))
            def _():
                pltpu.make_async_copy(table_hbm.at[pl.ds(0, chunk)], rows_b,
                                      sem_b).wait()
                pltpu.sync_copy(rows_b, out_hbm.at[pl.ds(base + i * chunk,
                                                         chunk)])

            return carry

        lax.fori_loop(0, n_chunks, body, 0)

    return gather_kernel(table, idx_flat)


_BLOCK_ROWS = 2048


def _tc_fused_ln_slab(acc, embeds2, pos_j, tids3, type_table, w2, b2,
                      slab, slab_blocks):
    n, h = embeds2.shape
    br = _BLOCK_ROWS

    def body(*refs):
        if acc is None:
            emb_ref, pos_ref, tid_ref, tt_ref, w_ref, b_ref, out_ref = refs
        else:
            _, emb_ref, pos_ref, tid_ref, tt_ref, w_ref, b_ref, out_ref = refs
        e = emb_ref[...] + pos_ref[...]
        t = tid_ref[0, 0, :].reshape(br, 1)
        te = jnp.where(t == 1, tt_ref[1, :][None, :], tt_ref[0, :][None, :])
        e = e + te
        u = jnp.mean(e, axis=-1, keepdims=True)
        d = e - u
        s = jnp.mean(d * d, axis=-1, keepdims=True)
        x = d * lax.rsqrt(s + EPS)
        out_ref[...] = w_ref[...] * x + b_ref[...]

    in_specs = [
        pl.BlockSpec((br, h), lambda i, slab=slab: (slab * slab_blocks + i, 0)),
        pl.BlockSpec((br, h), lambda i: (i, 0)),
        pl.BlockSpec((1, 1, br), lambda i, slab=slab: (slab * slab_blocks + i, 0, 0)),
        pl.BlockSpec((2, h), lambda i: (0, 0)),
        pl.BlockSpec((1, h), lambda i: (0, 0)),
        pl.BlockSpec((1, h), lambda i: (0, 0)),
    ]
    args = [embeds2, pos_j, tids3, type_table, w2, b2]
    aliases = {}
    if acc is not None:
        in_specs = [pl.BlockSpec(memory_space=pl.ANY)] + in_specs
        args = [acc] + args
        aliases = {0: 0}

    return pl.pallas_call(
        body,
        grid=(slab_blocks,),
        in_specs=in_specs,
        out_specs=pl.BlockSpec(
            (br, h), lambda i, slab=slab: (slab * slab_blocks + i, 0)),
        out_shape=jax.ShapeDtypeStruct((n, h), jnp.float32),
        input_output_aliases=aliases,
    )(*args)


def kernel(inputs_embeds, token_type_ids, position_ids, pos_table,
           type_table, ln_weight, ln_bias):
    b, s, h = inputs_embeds.shape
    n = b * s
    embeds2 = inputs_embeds.reshape(n, h)
    pos_flat = position_ids.reshape(n).astype(jnp.int32)
    tids3 = token_type_ids.reshape(n // _BLOCK_ROWS, 1, _BLOCK_ROWS).astype(jnp.int32)
    w2 = ln_weight.reshape(1, h)
    b2 = ln_bias.reshape(1, h)

    slab_blocks = n // _BLOCK_ROWS
    pos2 = _sc_gather(pos_table, pos_flat)
    acc = _tc_fused_ln_slab(None, embeds2, pos2, tids3, type_table,
                            w2, b2, 0, slab_blocks)
    return acc.reshape(b, s, h)
